# Initial kernel scaffold; baseline (speedup 1.0000x reference)
import jax
import jax.numpy as jnp
from jax.experimental import pallas as pl


def kernel(sparse_features, table):
    raise NotImplementedError("write your pallas kernel here")


# speedup vs baseline: 1.2995x; 1.2995x over previous
import functools

import jax
import jax.numpy as jnp
from jax import lax
from jax.experimental import pallas as pl
from jax.experimental.pallas import tpu as pltpu
from jax.experimental.pallas import tpu_sc as plsc

B = 4096
F = 26
D = 128
VOCAB_PER_FIELD = 100000

NC = 2
NS = 16
L = 16
NW = NC * NS
BPW = B // NW
FLAT = BPW * F
C = 4
ROWS = C * F
NCHUNK = BPW // C

_mesh = plsc.VectorSubcoreMesh(core_axis_name="c", subcore_axis_name="s")


@functools.partial(
    pl.kernel,
    out_type=jax.ShapeDtypeStruct((B, D), jnp.float32),
    mesh=_mesh,
    scratch_types=[
        pltpu.VMEM((FLAT,), jnp.int32),
        pltpu.VMEM((2, ROWS, D), jnp.float32),
        pltpu.VMEM((BPW, D), jnp.float32),
        pltpu.SemaphoreType.DMA,
        pltpu.SemaphoreType.DMA,
    ],
)
def _emb_bag(sf_hbm, table_hbm, out_hbm, idx_v, rows_v, out_v, sem0, sem1):
    wid = lax.axis_index("s") * NC + lax.axis_index("c")
    base = wid * FLAT

    pltpu.sync_copy(sf_hbm.at[pl.ds(base, FLAT)], idx_v)

    def fix(i, _):
        off = i * L
        pos = off + lax.iota(jnp.int32, L)
        f = lax.rem(pos, F)
        idx_v[pl.ds(off, L)] = idx_v[pl.ds(off, L)] + f * VOCAB_PER_FIELD
        return 0

    lax.fori_loop(0, FLAT // L, fix, 0)

    sems = (sem0, sem1)

    def fire(j, slot):
        pltpu.async_copy(
            table_hbm.at[idx_v.at[pl.ds(j * ROWS, ROWS)]],
            rows_v.at[slot],
            sems[slot],
        )

    def drain(j, slot):
        pltpu.make_async_copy(
            table_hbm.at[idx_v.at[pl.ds(j * ROWS, ROWS)]],
            rows_v.at[slot],
            sems[slot],
        ).wait()

    def accumulate(j, slot):
        for b in range(C):
            for d in range(D // L):
                dd = pl.ds(d * L, L)
                acc = rows_v[slot, b * F, dd]
                for f in range(1, F):
                    acc = acc + rows_v[slot, b * F + f, dd]
                out_v[j * C + b, dd] = acc * (1.0 / F)

    fire(0, 0)

    def pair(j2, _):
        j0 = 2 * j2
        fire(j0 + 1, 1)
        drain(j0, 0)
        accumulate(j0, 0)

        @pl.when(j0 + 2 < NCHUNK)
        def _():
            fire(j0 + 2, 0)

        drain(j0 + 1, 1)
        accumulate(j0 + 1, 1)
        return 0

    lax.fori_loop(0, NCHUNK // 2, pair, 0)

    pltpu.sync_copy(out_v, out_hbm.at[pl.ds(wid * BPW, BPW)])


def kernel(sparse_features, table):
    sf_flat = sparse_features.astype(jnp.int32).reshape(-1)
    return _emb_bag(sf_flat, table)


# speedup vs baseline: 2.1567x; 1.6596x over previous
import functools

import jax
import jax.numpy as jnp
from jax import lax
from jax.experimental import pallas as pl
from jax.experimental.pallas import tpu as pltpu
from jax.experimental.pallas import tpu_sc as plsc

B = 4096
F = 26
D = 128
VOCAB_PER_FIELD = 100000

NC = 2
NS = 16
L = 16
NW = NC * NS
BPW = B // NW
FLAT = BPW * F
C = 4
ROWS = C * F
NCHUNK = BPW // C

_mesh = plsc.VectorSubcoreMesh(core_axis_name="c", subcore_axis_name="s")


@functools.partial(
    pl.kernel,
    out_type=jax.ShapeDtypeStruct((B, D), jnp.float32),
    mesh=_mesh,
    scratch_types=[
        pltpu.VMEM((FLAT,), jnp.int32),
        pltpu.VMEM((2, ROWS, D), jnp.float32),
        pltpu.VMEM((BPW, D), jnp.float32),
        pltpu.SemaphoreType.DMA,
        pltpu.SemaphoreType.DMA,
    ],
)
def _emb_bag(sf_hbm, table_hbm, out_hbm, idx_v, rows_v, out_v, sem0, sem1):
    wid = lax.axis_index("s") * NC + lax.axis_index("c")
    base = wid * FLAT

    pltpu.sync_copy(sf_hbm.at[pl.ds(base, FLAT)], idx_v)

    def fix(i, _):
        off = i * L
        pos = off + lax.iota(jnp.int32, L)
        f = lax.rem(pos, F)
        idx_v[pl.ds(off, L)] = idx_v[pl.ds(off, L)] + f * VOCAB_PER_FIELD
        return 0

    lax.fori_loop(0, FLAT // L, fix, 0)

    sems = (sem0, sem1)

    def fire(j, slot):
        pltpu.async_copy(
            table_hbm.at[idx_v.at[pl.ds(j * ROWS, ROWS)]],
            rows_v.at[slot],
            sems[slot],
        )

    def drain(j, slot):
        pltpu.make_async_copy(
            table_hbm.at[idx_v.at[pl.ds(j * ROWS, ROWS)]],
            rows_v.at[slot],
            sems[slot],
        ).wait()

    def accumulate(j, slot):
        G = 4
        for b in range(C):
            for d0 in range(0, D // L, G):
                ds_ = [pl.ds((d0 + g) * L, L) for g in range(G)]
                accs = [rows_v[slot, b * F, dd] for dd in ds_]
                for f in range(1, F):
                    for g in range(G):
                        accs[g] = accs[g] + rows_v[slot, b * F + f, ds_[g]]
                for g in range(G):
                    out_v[j * C + b, ds_[g]] = accs[g] * (1.0 / F)

    fire(0, 0)

    def pair(j2, _):
        j0 = 2 * j2
        fire(j0 + 1, 1)
        drain(j0, 0)
        accumulate(j0, 0)

        @pl.when(j0 + 2 < NCHUNK)
        def _():
            fire(j0 + 2, 0)

        drain(j0 + 1, 1)
        accumulate(j0 + 1, 1)
        return 0

    lax.fori_loop(0, NCHUNK // 2, pair, 0)

    pltpu.sync_copy(out_v, out_hbm.at[pl.ds(wid * BPW, BPW)])


def kernel(sparse_features, table):
    sf_flat = sparse_features.astype(jnp.int32).reshape(-1)
    return _emb_bag(sf_flat, table)
